# baseline (speedup 1.0000x reference)
import jax
import jax.numpy as jnp
from jax.experimental import pallas as pl
from jax.experimental.pallas import tpu as pltpu

D_MODEL_C = 1024
N_EXPERTS_C = 8
D_EXPERT_C = 512
N_ROUTED_C = 7
S_C = 2048
TM = 512
NF = 2
FH = D_EXPERT_C // NF


def _moe_body(x_ref, si_ref, keys_ref, values_ref, est_ref, bias_ref,
              out_ref, sel_ref, w_ref, acc_ref):
    f = pl.program_id(0)
    i = pl.program_id(1)
    tok = i * TM

    @pl.when(f == 0)
    def _routing():
        aff = jax.nn.sigmoid(
            jnp.dot(si_ref[...], est_ref[...],
                    preferred_element_type=jnp.float32))
        routed = aff[:, :N_ROUTED_C] + bias_ref[0, :N_ROUTED_C]
        iota7 = jax.lax.broadcasted_iota(jnp.int32, (TM, N_ROUTED_C), 1)
        m1 = jnp.max(routed, axis=1, keepdims=True)
        i1 = jnp.min(jnp.where(routed == m1, iota7, N_ROUTED_C), axis=1,
                     keepdims=True)
        masked = jnp.where(iota7 == i1, -jnp.inf, routed)
        m2 = jnp.max(masked, axis=1, keepdims=True)
        i2 = jnp.min(jnp.where(masked == m2, iota7, N_ROUTED_C), axis=1,
                     keepdims=True)
        iota8 = jax.lax.broadcasted_iota(jnp.int32, (TM, N_EXPERTS_C), 1)
        selmask = (iota8 == i1) | (iota8 == i2) | (iota8 == N_ROUTED_C)
        w_ref[pl.ds(tok, TM), :] = jnp.where(selmask, aff, 0.0)
        sel_ref[pl.ds(tok, TM), :] = jnp.concatenate(
            [i1, i2, jnp.full((TM, 1), N_ROUTED_C, jnp.int32)], axis=1)

    w = w_ref[pl.ds(tok, TM), :]

    xb = x_ref[...]
    acc = jnp.zeros((TM, D_MODEL_C), jnp.float32)
    for e in range(N_EXPERTS_C):
        h = jnp.dot(xb, keys_ref[e], preferred_element_type=jnp.float32)
        h = h * jax.nn.sigmoid(h)
        h = h * w[:, e:e + 1]
        acc = acc + jnp.dot(h, values_ref[e],
                            preferred_element_type=jnp.float32)

    @pl.when(f == 0)
    def _store_partial():
        acc_ref[pl.ds(tok, TM), :] = acc
        out_ref[...] = acc

    @pl.when(f == NF - 1)
    def _emit():
        out_ref[...] = acc_ref[pl.ds(tok, TM), :] + acc


def kernel(token_stream, selection_input, keys_w, values_w, expert_sel,
           bias_ffn):
    b, s, d = token_stream.shape
    x = token_stream.reshape(s, d)
    si = selection_input.reshape(s, d)
    est = expert_sel.T
    bias = bias_ffn.reshape(1, N_EXPERTS_C)

    grid = (NF, s // TM)
    out, sel = pl.pallas_call(
        _moe_body,
        grid=grid,
        in_specs=[
            pl.BlockSpec((TM, d), lambda f, i: (i, 0)),
            pl.BlockSpec((TM, d),
                         lambda f, i: (jnp.where(f == 0, i, s // TM - 1), 0)),
            pl.BlockSpec((N_EXPERTS_C, d, FH), lambda f, i: (0, 0, f)),
            pl.BlockSpec((N_EXPERTS_C, FH, d), lambda f, i: (0, f, 0)),
            pl.BlockSpec((d, N_EXPERTS_C), lambda f, i: (0, 0)),
            pl.BlockSpec((1, N_EXPERTS_C), lambda f, i: (0, 0)),
        ],
        out_specs=[
            pl.BlockSpec((TM, d), lambda f, i: (i, 0)),
            pl.BlockSpec((s, 3), lambda f, i: (0, 0)),
        ],
        out_shape=[
            jax.ShapeDtypeStruct((s, d), jnp.float32),
            jax.ShapeDtypeStruct((s, 3), jnp.int32),
        ],
        scratch_shapes=[
            pltpu.VMEM((s, N_EXPERTS_C), jnp.float32),
            pltpu.VMEM((s, d), jnp.float32),
        ],
    )(x, si, keys_w, values_w, est, bias)
    return out.reshape(b, s, d), sel.reshape(b, s, 3)


# speedup vs baseline: 1.1458x; 1.1458x over previous
import jax
import jax.numpy as jnp
from jax.experimental import pallas as pl

D_MODEL_C = 1024
N_EXPERTS_C = 8
D_EXPERT_C = 512
N_ROUTED_C = 7
TM = 512


def _moe_body(x_ref, si_ref, k0_ref, k1_ref, k2_ref, k3_ref,
              v0_ref, v1_ref, v2_ref, v3_ref, est_ref, bias_ref,
              out_ref, sel_ref):
    kparts = (k0_ref, k1_ref, k2_ref, k3_ref)
    vparts = (v0_ref, v1_ref, v2_ref, v3_ref)
    aff = jax.nn.sigmoid(
        jnp.dot(si_ref[...], est_ref[...], preferred_element_type=jnp.float32)
    )
    routed = aff[:, :N_ROUTED_C] + bias_ref[0, :N_ROUTED_C]
    iota7 = jax.lax.broadcasted_iota(jnp.int32, (TM, N_ROUTED_C), 1)
    m1 = jnp.max(routed, axis=1, keepdims=True)
    i1 = jnp.min(jnp.where(routed == m1, iota7, N_ROUTED_C), axis=1,
                 keepdims=True)
    masked = jnp.where(iota7 == i1, -jnp.inf, routed)
    m2 = jnp.max(masked, axis=1, keepdims=True)
    i2 = jnp.min(jnp.where(masked == m2, iota7, N_ROUTED_C), axis=1,
                 keepdims=True)
    iota8 = jax.lax.broadcasted_iota(jnp.int32, (TM, N_EXPERTS_C), 1)
    selmask = (iota8 == i1) | (iota8 == i2) | (iota8 == N_ROUTED_C)
    w = jnp.where(selmask, aff, 0.0)
    sel_ref[...] = jnp.concatenate(
        [i1, i2, jnp.full((TM, 1), N_ROUTED_C, jnp.int32)], axis=1)

    xb = x_ref[...]
    acc = jnp.zeros((TM, D_MODEL_C), jnp.float32)
    for e in range(N_EXPERTS_C):
        h = jnp.dot(xb, kparts[e // 2][e % 2],
                    preferred_element_type=jnp.float32)
        h = h * jax.nn.sigmoid(h)
        h = h * w[:, e:e + 1]
        acc = acc + jnp.dot(h, vparts[e // 2][e % 2],
                            preferred_element_type=jnp.float32)
    out_ref[...] = acc


def kernel(token_stream, selection_input, keys_w, values_w, expert_sel,
           bias_ffn):
    b, s, d = token_stream.shape
    x = token_stream.reshape(s, d)
    si = selection_input.reshape(s, d)
    est = expert_sel.T
    bias = bias_ffn.reshape(1, N_EXPERTS_C)

    grid = (s // TM,)
    out, sel = pl.pallas_call(
        _moe_body,
        grid=grid,
        in_specs=[
            pl.BlockSpec((TM, d), lambda i: (i, 0)),
            pl.BlockSpec((TM, d), lambda i: (i, 0)),
        ] + [
            pl.BlockSpec((2, d, D_EXPERT_C), lambda i, j=j: (j, 0, 0))
            for j in range(4)
        ] + [
            pl.BlockSpec((2, D_EXPERT_C, d), lambda i, j=j: (j, 0, 0))
            for j in range(4)
        ] + [
            pl.BlockSpec((d, N_EXPERTS_C), lambda i: (0, 0)),
            pl.BlockSpec((1, N_EXPERTS_C), lambda i: (0, 0)),
        ],
        out_specs=[
            pl.BlockSpec((TM, d), lambda i: (i, 0)),
            pl.BlockSpec((TM, 3), lambda i: (i, 0)),
        ],
        out_shape=[
            jax.ShapeDtypeStruct((s, d), jnp.float32),
            jax.ShapeDtypeStruct((s, 3), jnp.int32),
        ],
    )(x, si, keys_w, keys_w, keys_w, keys_w,
      values_w, values_w, values_w, values_w, est, bias)
    return out.reshape(b, s, d), sel.reshape(b, s, 3)
